# baseline (device time: 36668 ns/iter reference)
import jax
import jax.numpy as jnp
from jax import lax
from jax.experimental import pallas as pl
from jax.experimental.pallas import tpu as pltpu

N_DEV = 32
CHUNK = 512 // N_DEV


def kernel(x, W1, W2):
    m, k = x.shape
    h_per = W1.shape[1]
    n = W2.shape[1]

    def body(x_ref, w1_ref, w2_ref, out_ref,
             send_stage, rs_recv_buf, ag_buf,
             rs_send, rs_recv, ag_send, ag_recv):
        my = lax.axis_index("i")

        xb = x_ref[...].astype(jnp.bfloat16)
        w1b = w1_ref[...].astype(jnp.bfloat16)
        w2b = w2_ref[...].astype(jnp.bfloat16)
        h = jnp.maximum(
            jnp.dot(xb, w1b, preferred_element_type=jnp.float32),
            0.0,
        ).astype(jnp.bfloat16)
        partial = jnp.dot(h, w2b, preferred_element_type=jnp.float32)
        send_stage[...] = partial.astype(jnp.bfloat16)

        rs_sends = []
        for d in range(1, N_DEV):
            dest = (my + d) % N_DEV
            rdma = pltpu.make_async_remote_copy(
                src_ref=send_stage.at[pl.ds(dest * CHUNK, CHUNK), :],
                dst_ref=rs_recv_buf.at[d],
                send_sem=rs_send.at[d],
                recv_sem=rs_recv.at[d],
                device_id=(dest,),
                device_id_type=pl.DeviceIdType.MESH,
            )
            rdma.start()
            rs_sends.append(rdma)

        rs_recv_buf[0] = send_stage[pl.ds(my * CHUNK, CHUNK), :]
        for d in range(1, N_DEV):
            src_dev = (my - d) % N_DEV
            recv = pltpu.make_async_remote_copy(
                src_ref=send_stage.at[pl.ds(0, CHUNK), :],
                dst_ref=rs_recv_buf.at[d],
                send_sem=rs_send.at[d],
                recv_sem=rs_recv.at[d],
                device_id=(src_dev,),
                device_id_type=pl.DeviceIdType.MESH,
            )
            recv.wait_recv()

        reduced = jnp.sum(rs_recv_buf[...].astype(jnp.float32), axis=0)
        ag_buf[my] = reduced.astype(jnp.bfloat16)

        ag_sends = []
        for d in range(1, N_DEV):
            dest = (my + d) % N_DEV
            rdma = pltpu.make_async_remote_copy(
                src_ref=ag_buf.at[my],
                dst_ref=ag_buf.at[my],
                send_sem=ag_send.at[d],
                recv_sem=ag_recv.at[my],
                device_id=(dest,),
                device_id_type=pl.DeviceIdType.MESH,
            )
            rdma.start()
            ag_sends.append(rdma)

        for d in range(1, N_DEV):
            src_dev = (my - d) % N_DEV
            recv = pltpu.make_async_remote_copy(
                src_ref=send_stage.at[pl.ds(0, CHUNK), :],
                dst_ref=ag_buf.at[src_dev],
                send_sem=ag_send.at[d],
                recv_sem=ag_recv.at[src_dev],
                device_id=(src_dev,),
                device_id_type=pl.DeviceIdType.MESH,
            )
            recv.wait_recv()

        out_ref[...] = ag_buf[...].astype(jnp.float32).reshape(m, n)

        for rdma in rs_sends:
            rdma.wait_send()
        for rdma in ag_sends:
            rdma.wait_send()

    return pl.pallas_call(
        body,
        out_shape=jax.ShapeDtypeStruct((m, n), jnp.float32),
        in_specs=[
            pl.BlockSpec(memory_space=pltpu.VMEM),
            pl.BlockSpec(memory_space=pltpu.VMEM),
            pl.BlockSpec(memory_space=pltpu.VMEM),
        ],
        out_specs=pl.BlockSpec(memory_space=pltpu.VMEM),
        scratch_shapes=[
            pltpu.VMEM((m, n), jnp.bfloat16),
            pltpu.VMEM((N_DEV, CHUNK, n), jnp.bfloat16),
            pltpu.VMEM((N_DEV, CHUNK, n), jnp.bfloat16),
            pltpu.SemaphoreType.DMA((N_DEV,)),
            pltpu.SemaphoreType.DMA((N_DEV,)),
            pltpu.SemaphoreType.DMA((N_DEV,)),
            pltpu.SemaphoreType.DMA((N_DEV,)),
        ],
    )(x, W1, W2)


# device time: 28286 ns/iter; 1.2963x vs baseline; 1.2963x over previous
import jax
import jax.numpy as jnp
from jax import lax
from jax.experimental import pallas as pl
from jax.experimental.pallas import tpu as pltpu

N_DEV = 32
CHUNK = 512 // N_DEV


def kernel(x, W1, W2):
    m, k = x.shape
    h_per = W1.shape[1]
    n = W2.shape[1]

    def body(x_ref, w1_ref, w2_ref, out_ref,
             send_stage, rs_recv_buf, ag_buf,
             rs_send, rs_recv, ag_send, ag_recv):
        my = lax.axis_index("i")

        barrier_sem = pltpu.get_barrier_semaphore()
        for d in range(1, N_DEV):
            dest = (my + d) % N_DEV
            pl.semaphore_signal(
                barrier_sem, inc=1,
                device_id=(dest,), device_id_type=pl.DeviceIdType.MESH,
            )
        pl.semaphore_wait(barrier_sem, N_DEV - 1)

        xb = x_ref[...].astype(jnp.bfloat16)
        w1b = w1_ref[...].astype(jnp.bfloat16)
        w2b = w2_ref[...].astype(jnp.bfloat16)
        h = jnp.maximum(
            jnp.dot(xb, w1b, preferred_element_type=jnp.float32),
            0.0,
        ).astype(jnp.bfloat16)
        partial = jnp.dot(h, w2b, preferred_element_type=jnp.float32)
        send_stage[...] = partial.astype(jnp.bfloat16)

        rs_sends = []
        for d in range(1, N_DEV):
            dest = (my + d) % N_DEV
            rdma = pltpu.make_async_remote_copy(
                src_ref=send_stage.at[pl.ds(dest * CHUNK, CHUNK), :],
                dst_ref=rs_recv_buf.at[d],
                send_sem=rs_send.at[d],
                recv_sem=rs_recv.at[d],
                device_id=(dest,),
                device_id_type=pl.DeviceIdType.MESH,
            )
            rdma.start()
            rs_sends.append(rdma)

        rs_recv_buf[0] = send_stage[pl.ds(my * CHUNK, CHUNK), :]
        for d in range(1, N_DEV):
            src_dev = (my - d) % N_DEV
            recv = pltpu.make_async_remote_copy(
                src_ref=send_stage.at[pl.ds(0, CHUNK), :],
                dst_ref=rs_recv_buf.at[d],
                send_sem=rs_send.at[d],
                recv_sem=rs_recv.at[d],
                device_id=(src_dev,),
                device_id_type=pl.DeviceIdType.MESH,
            )
            recv.wait_recv()

        reduced = jnp.sum(rs_recv_buf[...].astype(jnp.float32), axis=0)
        ag_buf[my] = reduced.astype(jnp.bfloat16)

        ag_sends = []
        for d in range(1, N_DEV):
            dest = (my + d) % N_DEV
            rdma = pltpu.make_async_remote_copy(
                src_ref=ag_buf.at[my],
                dst_ref=ag_buf.at[my],
                send_sem=ag_send.at[d],
                recv_sem=ag_recv.at[my],
                device_id=(dest,),
                device_id_type=pl.DeviceIdType.MESH,
            )
            rdma.start()
            ag_sends.append(rdma)

        for d in range(1, N_DEV):
            src_dev = (my - d) % N_DEV
            recv = pltpu.make_async_remote_copy(
                src_ref=send_stage.at[pl.ds(0, CHUNK), :],
                dst_ref=ag_buf.at[src_dev],
                send_sem=ag_send.at[d],
                recv_sem=ag_recv.at[src_dev],
                device_id=(src_dev,),
                device_id_type=pl.DeviceIdType.MESH,
            )
            recv.wait_recv()

        out_ref[...] = ag_buf[...].astype(jnp.float32).reshape(m, n)

        for rdma in rs_sends:
            rdma.wait_send()
        for rdma in ag_sends:
            rdma.wait_send()

    return pl.pallas_call(
        body,
        out_shape=jax.ShapeDtypeStruct((m, n), jnp.float32),
        in_specs=[
            pl.BlockSpec(memory_space=pltpu.VMEM),
            pl.BlockSpec(memory_space=pltpu.VMEM),
            pl.BlockSpec(memory_space=pltpu.VMEM),
        ],
        out_specs=pl.BlockSpec(memory_space=pltpu.VMEM),
        scratch_shapes=[
            pltpu.VMEM((m, n), jnp.bfloat16),
            pltpu.VMEM((N_DEV, CHUNK, n), jnp.bfloat16),
            pltpu.VMEM((N_DEV, CHUNK, n), jnp.bfloat16),
            pltpu.SemaphoreType.DMA((N_DEV,)),
            pltpu.SemaphoreType.DMA((N_DEV,)),
            pltpu.SemaphoreType.DMA((N_DEV,)),
            pltpu.SemaphoreType.DMA((N_DEV,)),
        ],
        compiler_params=pltpu.CompilerParams(collective_id=0),
    )(x, W1, W2)
